# baseline (speedup 1.0000x reference)
import functools

import jax
import jax.numpy as jnp
from jax import lax
from jax.experimental import pallas as pl
from jax.experimental.pallas import tpu as pltpu
from jax.experimental.pallas import tpu_sc as plsc

N = 10000
E = 320000
D_IN = 128
D_HID = 128
D_OUT = 64

NC = 2
NS = 16
NW = NC * NS
EPW = E // NW
C = 128
NCH = 80
EPWP = NCH * C
SB = 16
NSB = NCH // SB
NP = 10240
RPT = NP // NS

_MESH = plsc.VectorSubcoreMesh(core_axis_name="c", subcore_axis_name="s")


def _make_propagate(D):
    @functools.partial(
        pl.kernel,
        out_type=jax.ShapeDtypeStruct((NC, NP, D), jnp.float32),
        mesh=_MESH,
        scratch_types=[
            pltpu.VMEM((SB, C), jnp.int32),
            pltpu.VMEM((SB, C), jnp.int32),
            pltpu.VMEM((C, D), jnp.float32),
            pltpu.VMEM((C, D), jnp.float32),
            pltpu.VMEM_SHARED((NP, D), jnp.float32),
            pltpu.SemaphoreType.DMA,
            pltpu.SemaphoreType.DMA,
        ],
    )
    def propagate(hs, src3, dst3, out, src_v, dst_v,
                  buf0, buf1, acc, sem0, sem1):
        cid = lax.axis_index("c")
        sid = lax.axis_index("s")
        wid = cid * NS + sid
        rb = sid * RPT

        def initz(i, carry):
            def initcol(k, carry2):
                buf0[i, pl.ds(k * 16, 16)] = jnp.zeros((16,), jnp.float32)
                return carry2
            return lax.fori_loop(0, D // 16, initcol, carry)

        lax.fori_loop(0, C, initz, 0)

        def initacc(b, carry):
            pltpu.sync_copy(buf0, acc.at[pl.ds(rb + b * C, C)])
            return carry

        lax.fori_loop(0, RPT // C, initacc, 0)
        plsc.subcore_barrier()

        def superblock(s, carry):
            pltpu.sync_copy(src3.at[wid, pl.ds(s * SB, SB)], src_v)
            pltpu.sync_copy(dst3.at[wid, pl.ds(s * SB, SB)], dst_v)
            pltpu.async_copy(hs.at[src_v.at[0]], buf0, sem0)
            pltpu.async_copy(hs.at[src_v.at[1]], buf1, sem1)

            def step(t, carry2):
                k = 2 * t
                pltpu.make_async_copy(hs.at[src_v.at[k]], buf0, sem0).wait()
                pltpu.sync_copy(buf0, acc.at[dst_v.at[k]], add=True)

                @pl.when(k + 2 < SB)
                def _():
                    pltpu.async_copy(hs.at[src_v.at[k + 2]], buf0, sem0)

                pltpu.make_async_copy(hs.at[src_v.at[k + 1]], buf1, sem1).wait()
                pltpu.sync_copy(buf1, acc.at[dst_v.at[k + 1]], add=True)

                @pl.when(k + 3 < SB)
                def _():
                    pltpu.async_copy(hs.at[src_v.at[k + 3]], buf1, sem1)

                return carry2

            return lax.fori_loop(0, SB // 2, step, carry)

        lax.fori_loop(0, NSB, superblock, 0)
        plsc.subcore_barrier()
        pltpu.sync_copy(acc.at[pl.ds(rb, RPT)], out.at[cid, pl.ds(rb, RPT)])

    return propagate


@functools.partial(
    pl.kernel,
    out_type=jax.ShapeDtypeStruct((NC, NP, D_HID), jnp.float32),
    mesh=_MESH,
    scratch_types=[
        pltpu.VMEM((SB, C), jnp.int32),
        pltpu.VMEM((C, D_HID), jnp.float32),
        pltpu.VMEM((64, D_HID), jnp.float32),
        pltpu.VMEM_SHARED((NP, D_HID), jnp.float32),
        pltpu.SemaphoreType.DMA,
    ],
)
def _degree_kernel(dst3, out, dst_v, ones_v, zbuf, acc, sem):
    cid = lax.axis_index("c")
    sid = lax.axis_index("s")
    wid = cid * NS + sid
    rb = sid * RPT

    def initz(i, carry):
        def initcol(k, carry2):
            zbuf[i, pl.ds(k * 16, 16)] = jnp.zeros((16,), jnp.float32)
            return carry2
        return lax.fori_loop(0, D_HID // 16, initcol, carry)

    lax.fori_loop(0, 64, initz, 0)

    def initacc(b, carry):
        pltpu.sync_copy(zbuf, acc.at[pl.ds(rb + b * 64, 64)])
        return carry

    lax.fori_loop(0, RPT // 64, initacc, 0)

    def initones(i, carry):
        def initcol(k, carry2):
            ones_v[i, pl.ds(k * 16, 16)] = jnp.full((16,), 1.0, jnp.float32)
            return carry2
        return lax.fori_loop(0, D_HID // 16, initcol, carry)

    lax.fori_loop(0, C, initones, 0)
    plsc.subcore_barrier()

    def superblock(s, carry):
        pltpu.sync_copy(dst3.at[wid, pl.ds(s * SB, SB)], dst_v)

        def fire(k, carry2):
            pltpu.async_copy(ones_v, acc.at[dst_v.at[k]], sem, add=True)
            return carry2

        lax.fori_loop(0, SB, fire, carry)

        def drain(k, carry2):
            pltpu.make_async_copy(ones_v, acc.at[dst_v.at[k]], sem).wait()
            return carry2

        return lax.fori_loop(0, SB, drain, carry)

    lax.fori_loop(0, NSB, superblock, 0)
    plsc.subcore_barrier()
    pltpu.sync_copy(acc.at[pl.ds(rb, RPT)], out.at[cid, pl.ds(rb, RPT)])


_BR = 2000


def _dinv_block(dp_ref):
    deg = dp_ref[0, :, 0:1] + dp_ref[1, :, 0:1] + 1.0
    return lax.rsqrt(deg)


def _tc_matmul_scale(degp, x, W):
    d_out = W.shape[1]

    def body(dp_ref, x_ref, w_ref, o_ref):
        o_ref[...] = _dinv_block(dp_ref) * jnp.dot(
            x_ref[...], w_ref[...], preferred_element_type=jnp.float32)

    return pl.pallas_call(
        body,
        grid=(N // _BR,),
        in_specs=[
            pl.BlockSpec((2, _BR, D_HID), lambda i: (0, i, 0)),
            pl.BlockSpec((_BR, x.shape[1]), lambda i: (i, 0)),
            pl.BlockSpec(W.shape, lambda i: (0, 0)),
        ],
        out_specs=pl.BlockSpec((_BR, d_out), lambda i: (i, 0)),
        out_shape=jax.ShapeDtypeStruct((N, d_out), jnp.float32),
    )(degp, x, W)


def _tc_mid(degp, p1, hs1, b1, W2p):
    def body(dp_ref, p_ref, hs_ref, b_ref, w_ref, o_ref):
        dinv = _dinv_block(dp_ref)
        t = (p_ref[0] + p_ref[1] + hs_ref[...]) * dinv + b_ref[...]
        h = jnp.maximum(t, 0.0)
        o_ref[...] = dinv * jnp.dot(h, w_ref[...],
                                    preferred_element_type=jnp.float32)

    return pl.pallas_call(
        body,
        grid=(N // _BR,),
        in_specs=[
            pl.BlockSpec((2, _BR, D_HID), lambda i: (0, i, 0)),
            pl.BlockSpec((2, _BR, D_HID), lambda i: (0, i, 0)),
            pl.BlockSpec((_BR, D_HID), lambda i: (i, 0)),
            pl.BlockSpec((1, D_HID), lambda i: (0, 0)),
            pl.BlockSpec((D_HID, D_HID), lambda i: (0, 0)),
        ],
        out_specs=pl.BlockSpec((_BR, D_HID), lambda i: (i, 0)),
        out_shape=jax.ShapeDtypeStruct((N, D_HID), jnp.float32),
    )(degp, p1, hs1, b1, W2p)


def _tc_out(degp, p2, hs2, b2p):
    def body(dp_ref, p_ref, hs_ref, b_ref, o_ref):
        dinv = _dinv_block(dp_ref)
        o = (p_ref[0] + p_ref[1] + hs_ref[...]) * dinv + b_ref[...]
        m = jnp.max(o, axis=1, keepdims=True)
        e = jnp.exp(o - m)
        lse = jnp.log(jnp.sum(e, axis=1, keepdims=True)) + m
        o_ref[...] = (o - lse)[:, :D_OUT]

    return pl.pallas_call(
        body,
        grid=(N // _BR,),
        in_specs=[
            pl.BlockSpec((2, _BR, D_HID), lambda i: (0, i, 0)),
            pl.BlockSpec((2, _BR, D_HID), lambda i: (0, i, 0)),
            pl.BlockSpec((_BR, D_HID), lambda i: (i, 0)),
            pl.BlockSpec((1, D_HID), lambda i: (0, 0)),
        ],
        out_specs=pl.BlockSpec((_BR, D_OUT), lambda i: (i, 0)),
        out_shape=jax.ShapeDtypeStruct((N, D_OUT), jnp.float32),
    )(degp, p2, hs2, b2p)


_propagate_128 = _make_propagate(D_HID)


@jax.jit
def kernel(x, edge_index, W1, b1, W2, b2):
    npad = EPWP - EPW
    pad_src = jnp.broadcast_to((jnp.arange(npad, dtype=jnp.int32) * 41) % N,
                               (NW, npad))
    pad_dst = jnp.broadcast_to(N + (jnp.arange(npad, dtype=jnp.int32)
                                    % (NP - N)), (NW, npad))
    src3 = jnp.concatenate(
        [edge_index[0].astype(jnp.int32).reshape(NW, EPW), pad_src],
        axis=1).reshape(NW, NCH, C)
    dst3 = jnp.concatenate(
        [edge_index[1].astype(jnp.int32).reshape(NW, EPW), pad_dst],
        axis=1).reshape(NW, NCH, C)
    W2p = jnp.pad(W2, ((0, 0), (0, D_HID - D_OUT)))
    b2p = jnp.concatenate(
        [b2, jnp.full((D_HID - D_OUT,), -1e30, jnp.float32)]).reshape(1, D_HID)

    degp = _degree_kernel(dst3)
    hs1 = _tc_matmul_scale(degp, x, W1)
    p1 = _propagate_128(hs1, src3, dst3)
    hs2 = _tc_mid(degp, p1, hs1, b1.reshape(1, D_HID), W2p)
    p2 = _propagate_128(hs2, src3, dst3)
    return _tc_out(degp, p2, hs2, b2p)


# speedup vs baseline: 1.0543x; 1.0543x over previous
import functools

import jax
import jax.numpy as jnp
from jax import lax
from jax.experimental import pallas as pl
from jax.experimental.pallas import tpu as pltpu
from jax.experimental.pallas import tpu_sc as plsc

N = 10000
E = 320000
D_IN = 128
D_HID = 128
D_OUT = 64

NC = 2
NS = 16
NW = NC * NS
EPW = E // NW
C = 128
NCH = 80
EPWP = NCH * C
SB = 40
NSB = NCH // SB
NP = 10240
RPT = NP // NS

_MESH = plsc.VectorSubcoreMesh(core_axis_name="c", subcore_axis_name="s")


def _make_propagate(D):
    @functools.partial(
        pl.kernel,
        out_type=jax.ShapeDtypeStruct((NC, NP, D), jnp.float32),
        mesh=_MESH,
        scratch_types=[
            pltpu.VMEM((SB, C), jnp.int32),
            pltpu.VMEM((SB, C), jnp.int32),
            pltpu.VMEM((C, D), jnp.float32),
            pltpu.VMEM((C, D), jnp.float32),
            pltpu.VMEM_SHARED((NP, D), jnp.float32),
            pltpu.SemaphoreType.DMA,
            pltpu.SemaphoreType.DMA,
        ],
    )
    def propagate(hs, src3, dst3, out, src_v, dst_v,
                  buf0, buf1, acc, sem0, sem1):
        cid = lax.axis_index("c")
        sid = lax.axis_index("s")
        wid = cid * NS + sid
        rb = sid * RPT

        def initz(i, carry):
            def initcol(k, carry2):
                buf0[i, pl.ds(k * 16, 16)] = jnp.zeros((16,), jnp.float32)
                return carry2
            return lax.fori_loop(0, D // 16, initcol, carry)

        lax.fori_loop(0, C, initz, 0)

        def initacc(b, carry):
            pltpu.sync_copy(buf0, acc.at[pl.ds(rb + b * C, C)])
            return carry

        lax.fori_loop(0, RPT // C, initacc, 0)
        plsc.subcore_barrier()

        def superblock(s, carry):
            pltpu.sync_copy(src3.at[wid, pl.ds(s * SB, SB)], src_v)
            pltpu.sync_copy(dst3.at[wid, pl.ds(s * SB, SB)], dst_v)
            pltpu.async_copy(hs.at[src_v.at[0]], buf0, sem0)
            pltpu.async_copy(hs.at[src_v.at[1]], buf1, sem1)

            def step(t, carry2):
                k = 2 * t
                pltpu.make_async_copy(hs.at[src_v.at[k]], buf0, sem0).wait()
                pltpu.sync_copy(buf0, acc.at[dst_v.at[k]], add=True)

                @pl.when(k + 2 < SB)
                def _():
                    pltpu.async_copy(hs.at[src_v.at[k + 2]], buf0, sem0)

                pltpu.make_async_copy(hs.at[src_v.at[k + 1]], buf1, sem1).wait()
                pltpu.sync_copy(buf1, acc.at[dst_v.at[k + 1]], add=True)

                @pl.when(k + 3 < SB)
                def _():
                    pltpu.async_copy(hs.at[src_v.at[k + 3]], buf1, sem1)

                return carry2

            return lax.fori_loop(0, SB // 2, step, carry)

        lax.fori_loop(0, NSB, superblock, 0)
        plsc.subcore_barrier()
        pltpu.sync_copy(acc.at[pl.ds(rb, RPT)], out.at[cid, pl.ds(rb, RPT)])

    return propagate


@functools.partial(
    pl.kernel,
    out_type=jax.ShapeDtypeStruct((NC, NP, D_HID), jnp.float32),
    mesh=_MESH,
    scratch_types=[
        pltpu.VMEM((SB, C), jnp.int32),
        pltpu.VMEM((C, D_HID), jnp.float32),
        pltpu.VMEM((64, D_HID), jnp.float32),
        pltpu.VMEM_SHARED((NP, D_HID), jnp.float32),
        pltpu.SemaphoreType.DMA,
    ],
)
def _degree_kernel(dst3, out, dst_v, ones_v, zbuf, acc, sem):
    cid = lax.axis_index("c")
    sid = lax.axis_index("s")
    wid = cid * NS + sid
    rb = sid * RPT

    def initz(i, carry):
        def initcol(k, carry2):
            zbuf[i, pl.ds(k * 16, 16)] = jnp.zeros((16,), jnp.float32)
            return carry2
        return lax.fori_loop(0, D_HID // 16, initcol, carry)

    lax.fori_loop(0, 64, initz, 0)

    def initacc(b, carry):
        pltpu.sync_copy(zbuf, acc.at[pl.ds(rb + b * 64, 64)])
        return carry

    lax.fori_loop(0, RPT // 64, initacc, 0)

    def initones(i, carry):
        def initcol(k, carry2):
            ones_v[i, pl.ds(k * 16, 16)] = jnp.full((16,), 1.0, jnp.float32)
            return carry2
        return lax.fori_loop(0, D_HID // 16, initcol, carry)

    lax.fori_loop(0, C, initones, 0)
    plsc.subcore_barrier()

    def superblock(s, carry):
        pltpu.sync_copy(dst3.at[wid, pl.ds(s * SB, SB)], dst_v)

        def fire(k, carry2):
            pltpu.async_copy(ones_v, acc.at[dst_v.at[k]], sem, add=True)
            return carry2

        lax.fori_loop(0, SB, fire, carry)

        def drain(k, carry2):
            pltpu.make_async_copy(ones_v, acc.at[dst_v.at[k]], sem).wait()
            return carry2

        return lax.fori_loop(0, SB, drain, carry)

    lax.fori_loop(0, NSB, superblock, 0)
    plsc.subcore_barrier()
    pltpu.sync_copy(acc.at[pl.ds(rb, RPT)], out.at[cid, pl.ds(rb, RPT)])


_BR = 2000


def _dinv_block(dp_ref):
    deg = dp_ref[0, :, 0:1] + dp_ref[1, :, 0:1] + 1.0
    return lax.rsqrt(deg)


def _tc_matmul_scale(degp, x, W):
    d_out = W.shape[1]

    def body(dp_ref, x_ref, w_ref, o_ref, dinv_ref):
        dinv = _dinv_block(dp_ref)
        dinv_ref[...] = jnp.broadcast_to(dinv, (dinv.shape[0], 8))
        o_ref[...] = dinv * jnp.dot(
            x_ref[...], w_ref[...], preferred_element_type=jnp.float32)

    return pl.pallas_call(
        body,
        grid=(N // _BR,),
        in_specs=[
            pl.BlockSpec((2, _BR, D_HID), lambda i: (0, i, 0)),
            pl.BlockSpec((_BR, x.shape[1]), lambda i: (i, 0)),
            pl.BlockSpec(W.shape, lambda i: (0, 0)),
        ],
        out_specs=[
            pl.BlockSpec((_BR, d_out), lambda i: (i, 0)),
            pl.BlockSpec((_BR, 8), lambda i: (i, 0)),
        ],
        out_shape=[
            jax.ShapeDtypeStruct((N, d_out), jnp.float32),
            jax.ShapeDtypeStruct((N, 8), jnp.float32),
        ],
    )(degp, x, W)


def _tc_mid(dinv8, p1, hs1, b1, W2p):
    def body(dv_ref, p_ref, hs_ref, b_ref, w_ref, o_ref):
        dinv = dv_ref[:, 0:1]
        t = (p_ref[0] + p_ref[1] + hs_ref[...]) * dinv + b_ref[...]
        h = jnp.maximum(t, 0.0)
        o_ref[...] = dinv * jnp.dot(h, w_ref[...],
                                    preferred_element_type=jnp.float32)

    return pl.pallas_call(
        body,
        grid=(N // _BR,),
        in_specs=[
            pl.BlockSpec((_BR, 8), lambda i: (i, 0)),
            pl.BlockSpec((2, _BR, D_HID), lambda i: (0, i, 0)),
            pl.BlockSpec((_BR, D_HID), lambda i: (i, 0)),
            pl.BlockSpec((1, D_HID), lambda i: (0, 0)),
            pl.BlockSpec((D_HID, D_HID), lambda i: (0, 0)),
        ],
        out_specs=pl.BlockSpec((_BR, D_HID), lambda i: (i, 0)),
        out_shape=jax.ShapeDtypeStruct((N, D_HID), jnp.float32),
    )(dinv8, p1, hs1, b1, W2p)


def _tc_out(dinv8, p2, hs2, b2p):
    def body(dv_ref, p_ref, hs_ref, b_ref, o_ref):
        dinv = dv_ref[:, 0:1]
        o = (p_ref[0] + p_ref[1] + hs_ref[...]) * dinv + b_ref[...]
        m = jnp.max(o, axis=1, keepdims=True)
        e = jnp.exp(o - m)
        lse = jnp.log(jnp.sum(e, axis=1, keepdims=True)) + m
        o_ref[...] = (o - lse)[:, :D_OUT]

    return pl.pallas_call(
        body,
        grid=(N // _BR,),
        in_specs=[
            pl.BlockSpec((_BR, 8), lambda i: (i, 0)),
            pl.BlockSpec((2, _BR, D_HID), lambda i: (0, i, 0)),
            pl.BlockSpec((_BR, D_HID), lambda i: (i, 0)),
            pl.BlockSpec((1, D_HID), lambda i: (0, 0)),
        ],
        out_specs=pl.BlockSpec((_BR, D_OUT), lambda i: (i, 0)),
        out_shape=jax.ShapeDtypeStruct((N, D_OUT), jnp.float32),
    )(dinv8, p2, hs2, b2p)


_propagate_128 = _make_propagate(D_HID)


@jax.jit
def kernel(x, edge_index, W1, b1, W2, b2):
    npad = EPWP - EPW
    pad_src = jnp.broadcast_to((jnp.arange(npad, dtype=jnp.int32) * 41) % N,
                               (NW, npad))
    pad_dst = jnp.broadcast_to(N + (jnp.arange(npad, dtype=jnp.int32)
                                    % (NP - N)), (NW, npad))
    src3 = jnp.concatenate(
        [edge_index[0].astype(jnp.int32).reshape(NW, EPW), pad_src],
        axis=1).reshape(NW, NCH, C)
    dst3 = jnp.concatenate(
        [edge_index[1].astype(jnp.int32).reshape(NW, EPW), pad_dst],
        axis=1).reshape(NW, NCH, C)
    W2p = jnp.pad(W2, ((0, 0), (0, D_HID - D_OUT)))
    b2p = jnp.concatenate(
        [b2, jnp.full((D_HID - D_OUT,), -1e30, jnp.float32)]).reshape(1, D_HID)

    degp = _degree_kernel(dst3)
    hs1, dinv8 = _tc_matmul_scale(degp, x, W1)
    p1 = _propagate_128(hs1, src3, dst3)
    hs2 = _tc_mid(dinv8, p1, hs1, b1.reshape(1, D_HID), W2p)
    p2 = _propagate_128(hs2, src3, dst3)
    return _tc_out(dinv8, p2, hs2, b2p)


# speedup vs baseline: 1.1368x; 1.0783x over previous
import functools

import jax
import jax.numpy as jnp
from jax import lax
from jax.experimental import pallas as pl
from jax.experimental.pallas import tpu as pltpu
from jax.experimental.pallas import tpu_sc as plsc

N = 10000
E = 320000
D_IN = 128
D_HID = 128
D_OUT = 64

NC = 2
NS = 16
NW = NC * NS
EPW = E // NW
C = 128
NCH = 80
EPWP = NCH * C
SB = 40
NSB = NCH // SB
NP = 10240
RPT = NP // NS

_MESH = plsc.VectorSubcoreMesh(core_axis_name="c", subcore_axis_name="s")


def _make_propagate(D, untiled=False):
    params = (pltpu.CompilerParams(use_tc_tiling_on_sc=False)
              if untiled else None)

    @functools.partial(
        pl.kernel,
        out_type=jax.ShapeDtypeStruct((NC, NP, D), jnp.float32),
        mesh=_MESH,
        compiler_params=params,
        scratch_types=[
            pltpu.VMEM((SB, C), jnp.int32),
            pltpu.VMEM((SB, C), jnp.int32),
            pltpu.VMEM((C, D), jnp.float32),
            pltpu.VMEM((C, D), jnp.float32),
            pltpu.VMEM_SHARED((NP, D), jnp.float32),
            pltpu.SemaphoreType.DMA,
            pltpu.SemaphoreType.DMA,
        ],
    )
    def propagate(hs, src3, dst3, out, src_v, dst_v,
                  buf0, buf1, acc, sem0, sem1):
        cid = lax.axis_index("c")
        sid = lax.axis_index("s")
        wid = cid * NS + sid
        rb = sid * RPT

        def initz(i, carry):
            def initcol(k, carry2):
                buf0[i, pl.ds(k * 16, 16)] = jnp.zeros((16,), jnp.float32)
                return carry2
            return lax.fori_loop(0, D // 16, initcol, carry)

        lax.fori_loop(0, C, initz, 0)

        def initacc(b, carry):
            pltpu.sync_copy(buf0, acc.at[pl.ds(rb + b * C, C)])
            return carry

        lax.fori_loop(0, RPT // C, initacc, 0)
        plsc.subcore_barrier()

        def superblock(s, carry):
            pltpu.sync_copy(src3.at[wid, pl.ds(s * SB, SB)], src_v)
            pltpu.sync_copy(dst3.at[wid, pl.ds(s * SB, SB)], dst_v)
            pltpu.async_copy(hs.at[src_v.at[0]], buf0, sem0)
            pltpu.async_copy(hs.at[src_v.at[1]], buf1, sem1)

            def step(t, carry2):
                k = 2 * t
                pltpu.make_async_copy(hs.at[src_v.at[k]], buf0, sem0).wait()
                pltpu.sync_copy(buf0, acc.at[dst_v.at[k]], add=True)

                @pl.when(k + 2 < SB)
                def _():
                    pltpu.async_copy(hs.at[src_v.at[k + 2]], buf0, sem0)

                pltpu.make_async_copy(hs.at[src_v.at[k + 1]], buf1, sem1).wait()
                pltpu.sync_copy(buf1, acc.at[dst_v.at[k + 1]], add=True)

                @pl.when(k + 3 < SB)
                def _():
                    pltpu.async_copy(hs.at[src_v.at[k + 3]], buf1, sem1)

                return carry2

            return lax.fori_loop(0, SB // 2, step, carry)

        lax.fori_loop(0, NSB, superblock, 0)
        plsc.subcore_barrier()
        pltpu.sync_copy(acc.at[pl.ds(rb, RPT)], out.at[cid, pl.ds(rb, RPT)])

    return propagate


@functools.partial(
    pl.kernel,
    out_type=jax.ShapeDtypeStruct((NC, NP, D_HID), jnp.float32),
    mesh=_MESH,
    scratch_types=[
        pltpu.VMEM((SB, C), jnp.int32),
        pltpu.VMEM((C, D_HID), jnp.float32),
        pltpu.VMEM((64, D_HID), jnp.float32),
        pltpu.VMEM_SHARED((NP, D_HID), jnp.float32),
        pltpu.SemaphoreType.DMA,
    ],
)
def _degree_kernel(dst3, out, dst_v, ones_v, zbuf, acc, sem):
    cid = lax.axis_index("c")
    sid = lax.axis_index("s")
    wid = cid * NS + sid
    rb = sid * RPT

    def initz(i, carry):
        def initcol(k, carry2):
            zbuf[i, pl.ds(k * 16, 16)] = jnp.zeros((16,), jnp.float32)
            return carry2
        return lax.fori_loop(0, D_HID // 16, initcol, carry)

    lax.fori_loop(0, 64, initz, 0)

    def initacc(b, carry):
        pltpu.sync_copy(zbuf, acc.at[pl.ds(rb + b * 64, 64)])
        return carry

    lax.fori_loop(0, RPT // 64, initacc, 0)

    def initones(i, carry):
        def initcol(k, carry2):
            ones_v[i, pl.ds(k * 16, 16)] = jnp.full((16,), 1.0, jnp.float32)
            return carry2
        return lax.fori_loop(0, D_HID // 16, initcol, carry)

    lax.fori_loop(0, C, initones, 0)
    plsc.subcore_barrier()

    def superblock(s, carry):
        pltpu.sync_copy(dst3.at[wid, pl.ds(s * SB, SB)], dst_v)

        def fire(k, carry2):
            pltpu.async_copy(ones_v, acc.at[dst_v.at[k]], sem, add=True)
            return carry2

        lax.fori_loop(0, SB, fire, carry)

        def drain(k, carry2):
            pltpu.make_async_copy(ones_v, acc.at[dst_v.at[k]], sem).wait()
            return carry2

        return lax.fori_loop(0, SB, drain, carry)

    lax.fori_loop(0, NSB, superblock, 0)
    plsc.subcore_barrier()
    pltpu.sync_copy(acc.at[pl.ds(rb, RPT)], out.at[cid, pl.ds(rb, RPT)])


_BR = 2000


def _dinv_block(dp_ref):
    deg = dp_ref[0, :, 0:1] + dp_ref[1, :, 0:1] + 1.0
    return lax.rsqrt(deg)


def _tc_matmul_scale(degp, x, W):
    d_out = W.shape[1]

    def body(dp_ref, x_ref, w_ref, o_ref, dinv_ref):
        dinv = _dinv_block(dp_ref)
        dinv_ref[...] = jnp.broadcast_to(dinv, (dinv.shape[0], 8))
        o_ref[...] = dinv * jnp.dot(
            x_ref[...], w_ref[...], preferred_element_type=jnp.float32)

    return pl.pallas_call(
        body,
        grid=(N // _BR,),
        in_specs=[
            pl.BlockSpec((2, _BR, D_HID), lambda i: (0, i, 0)),
            pl.BlockSpec((_BR, x.shape[1]), lambda i: (i, 0)),
            pl.BlockSpec(W.shape, lambda i: (0, 0)),
        ],
        out_specs=[
            pl.BlockSpec((_BR, d_out), lambda i: (i, 0)),
            pl.BlockSpec((_BR, 8), lambda i: (i, 0)),
        ],
        out_shape=[
            jax.ShapeDtypeStruct((N, d_out), jnp.float32),
            jax.ShapeDtypeStruct((N, 8), jnp.float32),
        ],
    )(degp, x, W)


def _tc_mid(dinv8, p1, hs1, b1, W2):
    def body(dv_ref, p_ref, hs_ref, b_ref, w_ref, o_ref):
        dinv = dv_ref[:, 0:1]
        t = (p_ref[0] + p_ref[1] + hs_ref[...]) * dinv + b_ref[...]
        h = jnp.maximum(t, 0.0)
        o_ref[...] = dinv * jnp.dot(h, w_ref[...],
                                    preferred_element_type=jnp.float32)

    return pl.pallas_call(
        body,
        grid=(N // _BR,),
        in_specs=[
            pl.BlockSpec((_BR, 8), lambda i: (i, 0)),
            pl.BlockSpec((2, _BR, D_HID), lambda i: (0, i, 0)),
            pl.BlockSpec((_BR, D_HID), lambda i: (i, 0)),
            pl.BlockSpec((1, D_HID), lambda i: (0, 0)),
            pl.BlockSpec((D_HID, D_OUT), lambda i: (0, 0)),
        ],
        out_specs=pl.BlockSpec((_BR, D_OUT), lambda i: (i, 0)),
        out_shape=jax.ShapeDtypeStruct((N, D_OUT), jnp.float32),
    )(dinv8, p1, hs1, b1, W2)


def _tc_out(dinv8, p2, hs2, b2):
    def body(dv_ref, p_ref, hs_ref, b_ref, o_ref):
        dinv = dv_ref[:, 0:1]
        o = (p_ref[0] + p_ref[1] + hs_ref[...]) * dinv + b_ref[...]
        m = jnp.max(o, axis=1, keepdims=True)
        e = jnp.exp(o - m)
        lse = jnp.log(jnp.sum(e, axis=1, keepdims=True)) + m
        o_ref[...] = o - lse

    return pl.pallas_call(
        body,
        grid=(N // _BR,),
        in_specs=[
            pl.BlockSpec((_BR, 8), lambda i: (i, 0)),
            pl.BlockSpec((2, _BR, D_OUT), lambda i: (0, i, 0)),
            pl.BlockSpec((_BR, D_OUT), lambda i: (i, 0)),
            pl.BlockSpec((1, D_OUT), lambda i: (0, 0)),
        ],
        out_specs=pl.BlockSpec((_BR, D_OUT), lambda i: (i, 0)),
        out_shape=jax.ShapeDtypeStruct((N, D_OUT), jnp.float32),
    )(dinv8, p2, hs2, b2)


_propagate_128 = _make_propagate(D_HID)
_propagate_64 = _make_propagate(D_OUT, untiled=True)


@jax.jit
def kernel(x, edge_index, W1, b1, W2, b2):
    npad = EPWP - EPW
    pad_src = jnp.broadcast_to((jnp.arange(npad, dtype=jnp.int32) * 41) % N,
                               (NW, npad))
    pad_dst = jnp.broadcast_to(N + (jnp.arange(npad, dtype=jnp.int32)
                                    % (NP - N)), (NW, npad))
    src3 = jnp.concatenate(
        [edge_index[0].astype(jnp.int32).reshape(NW, EPW), pad_src],
        axis=1).reshape(NW, NCH, C)
    dst3 = jnp.concatenate(
        [edge_index[1].astype(jnp.int32).reshape(NW, EPW), pad_dst],
        axis=1).reshape(NW, NCH, C)
    degp = _degree_kernel(dst3)
    hs1, dinv8 = _tc_matmul_scale(degp, x, W1)
    p1 = _propagate_128(hs1, src3, dst3)
    hs2 = _tc_mid(dinv8, p1, hs1, b1.reshape(1, D_HID), W2)
    p2 = _propagate_64(hs2, src3, dst3)
    return _tc_out(dinv8, p2, hs2, b2.reshape(1, D_OUT))


# speedup vs baseline: 1.3297x; 1.1696x over previous
import functools

import jax
import jax.numpy as jnp
from jax import lax
from jax.experimental import pallas as pl
from jax.experimental.pallas import tpu as pltpu
from jax.experimental.pallas import tpu_sc as plsc

N = 10000
E = 320000
D_IN = 128
D_HID = 128
D_OUT = 64

NC = 2
NS = 16
NW = NC * NS
EPW = E // NW
C = 128
NCH = 80
EPWP = NCH * C
SB = 40
NSB = NCH // SB
NP = 10240
RPT = NP // NS

_MESH = plsc.VectorSubcoreMesh(core_axis_name="c", subcore_axis_name="s")


def _make_propagate(D, untiled=False):
    params = (pltpu.CompilerParams(use_tc_tiling_on_sc=False)
              if untiled else None)

    @functools.partial(
        pl.kernel,
        out_type=jax.ShapeDtypeStruct((NC, NP, D), jnp.float32),
        mesh=_MESH,
        compiler_params=params,
        scratch_types=[
            pltpu.VMEM((SB, C), jnp.int32),
            pltpu.VMEM((SB, C), jnp.int32),
            pltpu.VMEM((C, D), jnp.float32),
            pltpu.VMEM((C, D), jnp.float32),
            pltpu.VMEM_SHARED((NP, D), jnp.float32),
            pltpu.SemaphoreType.DMA,
            pltpu.SemaphoreType.DMA,
        ],
    )
    def propagate(hs, src3, dst3, out, src_v, dst_v,
                  buf0, buf1, acc, sem0, sem1):
        cid = lax.axis_index("c")
        sid = lax.axis_index("s")
        wid = cid * NS + sid
        rb = sid * RPT

        def initz(i, carry):
            def initcol(k, carry2):
                buf0[i, pl.ds(k * 16, 16)] = jnp.zeros((16,), jnp.float32)
                return carry2
            return lax.fori_loop(0, D // 16, initcol, carry)

        lax.fori_loop(0, C, initz, 0)

        def initacc(b, carry):
            pltpu.sync_copy(buf0, acc.at[pl.ds(rb + b * C, C)])
            return carry

        lax.fori_loop(0, RPT // C, initacc, 0)
        plsc.subcore_barrier()

        def superblock(s, carry):
            pltpu.sync_copy(src3.at[wid, pl.ds(s * SB, SB)], src_v)
            pltpu.sync_copy(dst3.at[wid, pl.ds(s * SB, SB)], dst_v)
            pltpu.async_copy(hs.at[src_v.at[0]], buf0, sem0)
            pltpu.async_copy(hs.at[src_v.at[1]], buf1, sem1)

            def step(t, carry2):
                k = 2 * t
                pltpu.make_async_copy(hs.at[src_v.at[k]], buf0, sem0).wait()
                pltpu.sync_copy(buf0, acc.at[dst_v.at[k]], add=True)

                @pl.when(k + 2 < SB)
                def _():
                    pltpu.async_copy(hs.at[src_v.at[k + 2]], buf0, sem0)

                pltpu.make_async_copy(hs.at[src_v.at[k + 1]], buf1, sem1).wait()
                pltpu.sync_copy(buf1, acc.at[dst_v.at[k + 1]], add=True)

                @pl.when(k + 3 < SB)
                def _():
                    pltpu.async_copy(hs.at[src_v.at[k + 3]], buf1, sem1)

                return carry2

            return lax.fori_loop(0, SB // 2, step, carry)

        lax.fori_loop(0, NSB, superblock, 0)
        plsc.subcore_barrier()
        pltpu.sync_copy(acc.at[pl.ds(rb, RPT)], out.at[cid, pl.ds(rb, RPT)])

    return propagate


DW = 16


@functools.partial(
    pl.kernel,
    out_type=jax.ShapeDtypeStruct((NC, NP, DW), jnp.float32),
    mesh=_MESH,
    compiler_params=pltpu.CompilerParams(use_tc_tiling_on_sc=False),
    scratch_types=[
        pltpu.VMEM((SB, C), jnp.int32),
        pltpu.VMEM((C, DW), jnp.float32),
        pltpu.VMEM((64, DW), jnp.float32),
        pltpu.VMEM_SHARED((NP, DW), jnp.float32),
        pltpu.SemaphoreType.DMA,
    ],
)
def _degree_kernel(dst3, out, dst_v, ones_v, zbuf, acc, sem):
    cid = lax.axis_index("c")
    sid = lax.axis_index("s")
    wid = cid * NS + sid
    rb = sid * RPT

    def initz(i, carry):
        zbuf[i, pl.ds(0, 16)] = jnp.zeros((16,), jnp.float32)
        return carry

    lax.fori_loop(0, 64, initz, 0)

    def initacc(b, carry):
        pltpu.sync_copy(zbuf, acc.at[pl.ds(rb + b * 64, 64)])
        return carry

    lax.fori_loop(0, RPT // 64, initacc, 0)

    def initones(i, carry):
        ones_v[i, pl.ds(0, 16)] = jnp.full((16,), 1.0, jnp.float32)
        return carry

    lax.fori_loop(0, C, initones, 0)
    plsc.subcore_barrier()

    def superblock(s, carry):
        pltpu.sync_copy(dst3.at[wid, pl.ds(s * SB, SB)], dst_v)

        def fire(k, carry2):
            pltpu.async_copy(ones_v, acc.at[dst_v.at[k]], sem, add=True)
            return carry2

        lax.fori_loop(0, SB, fire, carry)

        def drain(k, carry2):
            pltpu.make_async_copy(ones_v, acc.at[dst_v.at[k]], sem).wait()
            return carry2

        return lax.fori_loop(0, SB, drain, carry)

    lax.fori_loop(0, NSB, superblock, 0)
    plsc.subcore_barrier()
    pltpu.sync_copy(acc.at[pl.ds(rb, RPT)], out.at[cid, pl.ds(rb, RPT)])


_BR = 2000


def _dinv_block(dp_ref):
    deg = dp_ref[0, :, 0:1] + dp_ref[1, :, 0:1] + 1.0
    return lax.rsqrt(deg)


def _tc_matmul_scale(degp, x, W):
    d_out = W.shape[1]

    def body(dp_ref, x_ref, w_ref, o_ref, dinv_ref):
        dinv = _dinv_block(dp_ref)
        dinv_ref[...] = jnp.broadcast_to(dinv, (dinv.shape[0], 8))
        o_ref[...] = dinv * jnp.dot(
            x_ref[...], w_ref[...], preferred_element_type=jnp.float32)

    return pl.pallas_call(
        body,
        grid=(N // _BR,),
        in_specs=[
            pl.BlockSpec((2, _BR, DW), lambda i: (0, i, 0)),
            pl.BlockSpec((_BR, x.shape[1]), lambda i: (i, 0)),
            pl.BlockSpec(W.shape, lambda i: (0, 0)),
        ],
        out_specs=[
            pl.BlockSpec((_BR, d_out), lambda i: (i, 0)),
            pl.BlockSpec((_BR, 8), lambda i: (i, 0)),
        ],
        out_shape=[
            jax.ShapeDtypeStruct((N, d_out), jnp.float32),
            jax.ShapeDtypeStruct((N, 8), jnp.float32),
        ],
    )(degp, x, W)


def _tc_mid(dinv8, p1, hs1, b1, W2):
    def body(dv_ref, p_ref, hs_ref, b_ref, w_ref, o_ref):
        dinv = dv_ref[:, 0:1]
        t = (p_ref[0] + p_ref[1] + hs_ref[...]) * dinv + b_ref[...]
        h = jnp.maximum(t, 0.0)
        o_ref[...] = dinv * jnp.dot(h, w_ref[...],
                                    preferred_element_type=jnp.float32)

    return pl.pallas_call(
        body,
        grid=(N // _BR,),
        in_specs=[
            pl.BlockSpec((_BR, 8), lambda i: (i, 0)),
            pl.BlockSpec((2, _BR, D_HID), lambda i: (0, i, 0)),
            pl.BlockSpec((_BR, D_HID), lambda i: (i, 0)),
            pl.BlockSpec((1, D_HID), lambda i: (0, 0)),
            pl.BlockSpec((D_HID, D_OUT), lambda i: (0, 0)),
        ],
        out_specs=pl.BlockSpec((_BR, D_OUT), lambda i: (i, 0)),
        out_shape=jax.ShapeDtypeStruct((N, D_OUT), jnp.float32),
    )(dinv8, p1, hs1, b1, W2)


def _tc_out(dinv8, p2, hs2, b2):
    def body(dv_ref, p_ref, hs_ref, b_ref, o_ref):
        dinv = dv_ref[:, 0:1]
        o = (p_ref[0] + p_ref[1] + hs_ref[...]) * dinv + b_ref[...]
        m = jnp.max(o, axis=1, keepdims=True)
        e = jnp.exp(o - m)
        lse = jnp.log(jnp.sum(e, axis=1, keepdims=True)) + m
        o_ref[...] = o - lse

    return pl.pallas_call(
        body,
        grid=(N // _BR,),
        in_specs=[
            pl.BlockSpec((_BR, 8), lambda i: (i, 0)),
            pl.BlockSpec((2, _BR, D_OUT), lambda i: (0, i, 0)),
            pl.BlockSpec((_BR, D_OUT), lambda i: (i, 0)),
            pl.BlockSpec((1, D_OUT), lambda i: (0, 0)),
        ],
        out_specs=pl.BlockSpec((_BR, D_OUT), lambda i: (i, 0)),
        out_shape=jax.ShapeDtypeStruct((N, D_OUT), jnp.float32),
    )(dinv8, p2, hs2, b2)


_propagate_128 = _make_propagate(D_HID)
_propagate_64 = _make_propagate(D_OUT, untiled=True)


@jax.jit
def kernel(x, edge_index, W1, b1, W2, b2):
    npad = EPWP - EPW
    pad_src = jnp.broadcast_to((jnp.arange(npad, dtype=jnp.int32) * 41) % N,
                               (NW, npad))
    pad_dst = jnp.broadcast_to(N + (jnp.arange(npad, dtype=jnp.int32)
                                    % (NP - N)), (NW, npad))
    src3 = jnp.concatenate(
        [edge_index[0].astype(jnp.int32).reshape(NW, EPW), pad_src],
        axis=1).reshape(NW, NCH, C)
    dst3 = jnp.concatenate(
        [edge_index[1].astype(jnp.int32).reshape(NW, EPW), pad_dst],
        axis=1).reshape(NW, NCH, C)
    degp = _degree_kernel(dst3)
    hs1, dinv8 = _tc_matmul_scale(degp, x, W1)
    p1 = _propagate_128(hs1, src3, dst3)
    hs2 = _tc_mid(dinv8, p1, hs1, b1.reshape(1, D_HID), W2)
    p2 = _propagate_64(hs2, src3, dst3)
    return _tc_out(dinv8, p2, hs2, b2.reshape(1, D_OUT))


# speedup vs baseline: 1.3440x; 1.0108x over previous
import functools

import jax
import jax.numpy as jnp
from jax import lax
from jax.experimental import pallas as pl
from jax.experimental.pallas import tpu as pltpu
from jax.experimental.pallas import tpu_sc as plsc

N = 10000
E = 320000
D_IN = 128
D_HID = 128
D_OUT = 64

NC = 2
NS = 16
NW = NC * NS
EPW = E // NW
C = 128
NCH = 80
EPWP = NCH * C
SB = 40
NSB = NCH // SB
NP = 10240
RPT = NP // NS

_MESH = plsc.VectorSubcoreMesh(core_axis_name="c", subcore_axis_name="s")


def _make_propagate(D, untiled=False, sb=SB):
    params = (pltpu.CompilerParams(use_tc_tiling_on_sc=False)
              if untiled else None)
    nsb = NCH // sb

    @functools.partial(
        pl.kernel,
        out_type=jax.ShapeDtypeStruct((NC, NP, D), jnp.float32),
        mesh=_MESH,
        compiler_params=params,
        scratch_types=[
            pltpu.VMEM((sb, C), jnp.int32),
            pltpu.VMEM((sb, C), jnp.int32),
            pltpu.VMEM((C, D), jnp.float32),
            pltpu.VMEM((C, D), jnp.float32),
            pltpu.VMEM_SHARED((NP, D), jnp.float32),
            pltpu.SemaphoreType.DMA,
            pltpu.SemaphoreType.DMA,
        ],
    )
    def propagate(hs, src3, dst3, out, src_v, dst_v,
                  buf0, buf1, acc, sem0, sem1):
        cid = lax.axis_index("c")
        sid = lax.axis_index("s")
        wid = cid * NS + sid
        rb = sid * RPT

        def initz(i, carry):
            def initcol(k, carry2):
                buf0[i, pl.ds(k * 16, 16)] = jnp.zeros((16,), jnp.float32)
                return carry2
            return lax.fori_loop(0, D // 16, initcol, carry)

        lax.fori_loop(0, C, initz, 0)

        def initacc(b, carry):
            pltpu.sync_copy(buf0, acc.at[pl.ds(rb + b * C, C)])
            return carry

        lax.fori_loop(0, RPT // C, initacc, 0)
        plsc.subcore_barrier()

        def superblock(s, carry):
            pltpu.sync_copy(src3.at[wid, pl.ds(s * sb, sb)], src_v)
            pltpu.sync_copy(dst3.at[wid, pl.ds(s * sb, sb)], dst_v)
            pltpu.async_copy(hs.at[src_v.at[0]], buf0, sem0)
            pltpu.async_copy(hs.at[src_v.at[1]], buf1, sem1)

            def step(t, carry2):
                k = 2 * t
                pltpu.make_async_copy(hs.at[src_v.at[k]], buf0, sem0).wait()
                pltpu.sync_copy(buf0, acc.at[dst_v.at[k]], add=True)

                @pl.when(k + 2 < sb)
                def _():
                    pltpu.async_copy(hs.at[src_v.at[k + 2]], buf0, sem0)

                pltpu.make_async_copy(hs.at[src_v.at[k + 1]], buf1, sem1).wait()
                pltpu.sync_copy(buf1, acc.at[dst_v.at[k + 1]], add=True)

                @pl.when(k + 3 < sb)
                def _():
                    pltpu.async_copy(hs.at[src_v.at[k + 3]], buf1, sem1)

                return carry2

            return lax.fori_loop(0, sb // 2, step, carry)

        lax.fori_loop(0, nsb, superblock, 0)
        plsc.subcore_barrier()
        pltpu.sync_copy(acc.at[pl.ds(rb, RPT)], out.at[cid, pl.ds(rb, RPT)])

    return propagate


DW = 16


@functools.partial(
    pl.kernel,
    out_type=jax.ShapeDtypeStruct((NC, NP, DW), jnp.float32),
    mesh=_MESH,
    compiler_params=pltpu.CompilerParams(use_tc_tiling_on_sc=False),
    scratch_types=[
        pltpu.VMEM((NCH, C), jnp.int32),
        pltpu.VMEM((C, DW), jnp.float32),
        pltpu.VMEM((64, DW), jnp.float32),
        pltpu.VMEM_SHARED((NP, DW), jnp.float32),
        pltpu.SemaphoreType.DMA,
    ],
)
def _degree_kernel(dst3, out, dst_v, ones_v, zbuf, acc, sem):
    cid = lax.axis_index("c")
    sid = lax.axis_index("s")
    wid = cid * NS + sid
    rb = sid * RPT

    def initz(i, carry):
        zbuf[i, pl.ds(0, 16)] = jnp.zeros((16,), jnp.float32)
        return carry

    lax.fori_loop(0, 64, initz, 0)

    def initacc(b, carry):
        pltpu.sync_copy(zbuf, acc.at[pl.ds(rb + b * 64, 64)])
        return carry

    lax.fori_loop(0, RPT // 64, initacc, 0)

    def initones(i, carry):
        ones_v[i, pl.ds(0, 16)] = jnp.full((16,), 1.0, jnp.float32)
        return carry

    lax.fori_loop(0, C, initones, 0)
    plsc.subcore_barrier()

    pltpu.sync_copy(dst3.at[wid], dst_v)

    def fire(k, carry):
        pltpu.async_copy(ones_v, acc.at[dst_v.at[k]], sem, add=True)
        return carry

    lax.fori_loop(0, NCH, fire, 0)

    def drain(k, carry):
        pltpu.make_async_copy(ones_v, acc.at[dst_v.at[k]], sem).wait()
        return carry

    lax.fori_loop(0, NCH, drain, 0)
    plsc.subcore_barrier()
    pltpu.sync_copy(acc.at[pl.ds(rb, RPT)], out.at[cid, pl.ds(rb, RPT)])


_BR = 2000


def _dinv_block(dp_ref):
    deg = dp_ref[0, :, 0:1] + dp_ref[1, :, 0:1] + 1.0
    return lax.rsqrt(deg)


def _tc_matmul_scale(degp, x, W):
    d_out = W.shape[1]

    def body(dp_ref, x_ref, w_ref, o_ref, dinv_ref):
        dinv = _dinv_block(dp_ref)
        dinv_ref[...] = jnp.broadcast_to(dinv, (dinv.shape[0], 8))
        o_ref[...] = dinv * jnp.dot(
            x_ref[...], w_ref[...], preferred_element_type=jnp.float32)

    return pl.pallas_call(
        body,
        grid=(N // _BR,),
        in_specs=[
            pl.BlockSpec((2, _BR, DW), lambda i: (0, i, 0)),
            pl.BlockSpec((_BR, x.shape[1]), lambda i: (i, 0)),
            pl.BlockSpec(W.shape, lambda i: (0, 0)),
        ],
        out_specs=[
            pl.BlockSpec((_BR, d_out), lambda i: (i, 0)),
            pl.BlockSpec((_BR, 8), lambda i: (i, 0)),
        ],
        out_shape=[
            jax.ShapeDtypeStruct((N, d_out), jnp.float32),
            jax.ShapeDtypeStruct((N, 8), jnp.float32),
        ],
    )(degp, x, W)


def _tc_mid(dinv8, p1, hs1, b1, W2):
    def body(dv_ref, p_ref, hs_ref, b_ref, w_ref, o_ref):
        dinv = dv_ref[:, 0:1]
        t = (p_ref[0] + p_ref[1] + hs_ref[...]) * dinv + b_ref[...]
        h = jnp.maximum(t, 0.0)
        o_ref[...] = dinv * jnp.dot(h, w_ref[...],
                                    preferred_element_type=jnp.float32)

    return pl.pallas_call(
        body,
        grid=(N // _BR,),
        in_specs=[
            pl.BlockSpec((_BR, 8), lambda i: (i, 0)),
            pl.BlockSpec((2, _BR, D_HID), lambda i: (0, i, 0)),
            pl.BlockSpec((_BR, D_HID), lambda i: (i, 0)),
            pl.BlockSpec((1, D_HID), lambda i: (0, 0)),
            pl.BlockSpec((D_HID, D_OUT), lambda i: (0, 0)),
        ],
        out_specs=pl.BlockSpec((_BR, D_OUT), lambda i: (i, 0)),
        out_shape=jax.ShapeDtypeStruct((N, D_OUT), jnp.float32),
    )(dinv8, p1, hs1, b1, W2)


def _tc_out(dinv8, p2, hs2, b2):
    def body(dv_ref, p_ref, hs_ref, b_ref, o_ref):
        dinv = dv_ref[:, 0:1]
        o = (p_ref[0] + p_ref[1] + hs_ref[...]) * dinv + b_ref[...]
        m = jnp.max(o, axis=1, keepdims=True)
        e = jnp.exp(o - m)
        lse = jnp.log(jnp.sum(e, axis=1, keepdims=True)) + m
        o_ref[...] = o - lse

    return pl.pallas_call(
        body,
        grid=(N // _BR,),
        in_specs=[
            pl.BlockSpec((_BR, 8), lambda i: (i, 0)),
            pl.BlockSpec((2, _BR, D_OUT), lambda i: (0, i, 0)),
            pl.BlockSpec((_BR, D_OUT), lambda i: (i, 0)),
            pl.BlockSpec((1, D_OUT), lambda i: (0, 0)),
        ],
        out_specs=pl.BlockSpec((_BR, D_OUT), lambda i: (i, 0)),
        out_shape=jax.ShapeDtypeStruct((N, D_OUT), jnp.float32),
    )(dinv8, p2, hs2, b2)


_propagate_128 = _make_propagate(D_HID)
_propagate_64 = _make_propagate(D_OUT, untiled=True, sb=NCH)


@jax.jit
def kernel(x, edge_index, W1, b1, W2, b2):
    npad = EPWP - EPW
    pad_src = jnp.broadcast_to((jnp.arange(npad, dtype=jnp.int32) * 41) % N,
                               (NW, npad))
    pad_dst = jnp.broadcast_to(N + (jnp.arange(npad, dtype=jnp.int32)
                                    % (NP - N)), (NW, npad))
    src3 = jnp.concatenate(
        [edge_index[0].astype(jnp.int32).reshape(NW, EPW), pad_src],
        axis=1).reshape(NW, NCH, C)
    dst3 = jnp.concatenate(
        [edge_index[1].astype(jnp.int32).reshape(NW, EPW), pad_dst],
        axis=1).reshape(NW, NCH, C)
    degp = _degree_kernel(dst3)
    hs1, dinv8 = _tc_matmul_scale(degp, x, W1)
    p1 = _propagate_128(hs1, src3, dst3)
    hs2 = _tc_mid(dinv8, p1, hs1, b1.reshape(1, D_HID), W2)
    p2 = _propagate_64(hs2, src3, dst3)
    return _tc_out(dinv8, p2, hs2, b2.reshape(1, D_OUT))
